# baseline (speedup 1.0000x reference)
import functools

import jax
import jax.numpy as jnp
from jax import lax
from jax.experimental import pallas as pl
from jax.experimental.pallas import tpu as pltpu
from jax.experimental.pallas import tpu_sc as plsc

F32 = jnp.float32
I32 = jnp.int32

N = 10000
E = 320000
K = 5000
EP = 160000
F = 128
H = 256

NS = 16
NPAD = 10240
KPAD = 5120


def _dot(a, b):
    return jnp.dot(a, b, preferred_element_type=F32)


def _half(x, j):
    return jnp.where(j == 0, x[:, :128], x[:, 128:])


def _t1_body(x_ref, w1, b1, w2, b2, wg, bg, xp_ref, h1_ref):
    j = pl.program_id(0)
    h = jnp.maximum(_dot(x_ref[...], w1[...]) + b1[...], 0.0)
    xp = jnp.maximum(_dot(h, w2[...]) + b2[...], 0.0)
    xp_ref[...] = _half(xp, j)
    h1_ref[...] = _dot(xp, wg[...]) + bg[...]


def _t1(x, W1, b1, W2, b2, Wg, bg):
    BM = 1000
    g = N // BM
    return pl.pallas_call(
        _t1_body,
        grid=(2, g),
        in_specs=[
            pl.BlockSpec((BM, F), lambda j, i: (i, 0)),
            pl.BlockSpec((F, H), lambda j, i: (0, 0)),
            pl.BlockSpec((1, H), lambda j, i: (0, 0)),
            pl.BlockSpec((H, H), lambda j, i: (0, 0)),
            pl.BlockSpec((1, H), lambda j, i: (0, 0)),
            pl.BlockSpec((H, 128), lambda j, i: (0, j)),
            pl.BlockSpec((1, 128), lambda j, i: (0, j)),
        ],
        out_specs=[
            pl.BlockSpec((BM, 128), lambda j, i: (j * g + i, 0)),
            pl.BlockSpec((BM, 128), lambda j, i: (j * g + i, 0)),
        ],
        out_shape=[
            jax.ShapeDtypeStruct((2 * N, 128), F32),
            jax.ShapeDtypeStruct((2 * N, 128), F32),
        ],
    )(x, W1, b1, W2, b2, Wg, bg)


def _t3_body(xp_ref, wgs, bgs, h_ref):
    h_ref[...] = _dot(xp_ref[...], wgs[...]) + bgs[...]


def _t3(x_pool, Wgs, bgs):
    BM = 1000
    g = K // BM
    return pl.pallas_call(
        _t3_body,
        grid=(2, g),
        in_specs=[
            pl.BlockSpec((BM, 2 * H), lambda j, i: (i, 0)),
            pl.BlockSpec((2 * H, 128), lambda j, i: (0, j)),
            pl.BlockSpec((1, 128), lambda j, i: (0, j)),
        ],
        out_specs=[pl.BlockSpec((BM, 128), lambda j, i: (j * g + i, 0))],
        out_shape=[jax.ShapeDtypeStruct((2 * K, 128), F32)],
    )(x_pool, Wgs, bgs)


def _t4_body(aga_ref, agb_ref, xp_ref, noise_ref, wg2, bg2, zm_ref, z_ref,
             h2_ref):
    gs = jnp.concatenate(
        [jnp.maximum(aga_ref[...], 0.0), jnp.maximum(agb_ref[...], 0.0)],
        axis=1)
    zm = jnp.concatenate([gs, xp_ref[...]], axis=1)
    z = zm + noise_ref[...] * jnp.exp(zm)
    zm_ref[...] = zm
    z_ref[...] = z
    h2_ref[...] = _dot(z, wg2[...]) + bg2[...]


def _t4(agg, x_pool, noise, Wg2, bg2):
    BM = 1000
    g = K // BM
    return pl.pallas_call(
        _t4_body,
        grid=(2, g),
        in_specs=[
            pl.BlockSpec((BM, 128), lambda j, i: (i, 0)),
            pl.BlockSpec((BM, 128), lambda j, i: (g + i, 0)),
            pl.BlockSpec((BM, 2 * H), lambda j, i: (i, 0)),
            pl.BlockSpec((BM, 3 * H), lambda j, i: (i, 0)),
            pl.BlockSpec((3 * H, 128), lambda j, i: (0, j)),
            pl.BlockSpec((1, 128), lambda j, i: (0, j)),
        ],
        out_specs=[
            pl.BlockSpec((BM, 3 * H), lambda j, i: (i, 0)),
            pl.BlockSpec((BM, 3 * H), lambda j, i: (i, 0)),
            pl.BlockSpec((BM, 128), lambda j, i: (j * g + i, 0)),
        ],
        out_shape=[
            jax.ShapeDtypeStruct((K, 3 * H), F32),
            jax.ShapeDtypeStruct((K, 3 * H), F32),
            jax.ShapeDtypeStruct((2 * K, 128), F32),
        ],
    )(agg, agg, x_pool, noise, Wg2, bg2)


def _t5_body(aga_ref, agb_ref, z_ref, wp1, bp1, wp2, bp2, xl_ref):
    g2 = jnp.concatenate(
        [jnp.maximum(aga_ref[...], 0.0), jnp.maximum(agb_ref[...], 0.0)],
        axis=1)
    zsk = jnp.concatenate([g2, z_ref[...]], axis=1)
    hh = jnp.maximum(_dot(zsk, wp1[...]) + bp1[...], 0.0)
    xl_ref[...] = _dot(hh, wp2[...]) + bp2[...]


def _t5(agg, z, Wp1, bp1, Wp2, bp2):
    BM = 1000
    g = K // BM
    return pl.pallas_call(
        _t5_body,
        grid=(g,),
        in_specs=[
            pl.BlockSpec((BM, 128), lambda i: (i, 0)),
            pl.BlockSpec((BM, 128), lambda i: (K // 1000 + i, 0)),
            pl.BlockSpec((BM, 3 * H), lambda i: (i, 0)),
            pl.BlockSpec((4 * H, H), lambda i: (0, 0)),
            pl.BlockSpec((1, H), lambda i: (0, 0)),
            pl.BlockSpec((H, F), lambda i: (0, 0)),
            pl.BlockSpec((1, F), lambda i: (0, 0)),
        ],
        out_specs=[pl.BlockSpec((BM, F), lambda i: (i, 0))],
        out_shape=[jax.ShapeDtypeStruct((K, F), F32)],
    )(agg, agg, z, Wp1, bp1, Wp2, bp2)


_MESH = plsc.VectorSubcoreMesh(core_axis_name="c", subcore_axis_name="s")


def _make_conv(num_edges, n_in, n_pad, n_out, passthrough, ch=40):
    ept = num_edges // NS
    rpt = n_pad // NS
    steps = ept // ch
    lastlen = n_out - 15 * rpt
    out_rows = (4 if passthrough else 2) * n_out

    scratch = [
        pltpu.VMEM((ch,), I32),
        pltpu.VMEM((ch,), I32),
        pltpu.VMEM((ch, 128), F32),
        pltpu.VMEM((ch,), I32),
        pltpu.VMEM((ch,), I32),
        pltpu.VMEM((ch, 128), F32),
        pltpu.VMEM_SHARED((n_pad, 128), F32),
        pltpu.SemaphoreType.DMA,
        pltpu.SemaphoreType.DMA,
        pltpu.SemaphoreType.DMA,
        pltpu.SemaphoreType.DMA,
    ]

    def body_pt(h_hbm, src_hbm, dst_hbm, zeros_hbm, xp_hbm, out_hbm,
                sidx0, didx0, rows0, sidx1, didx1, rows1, acc, sem0, sem1,
                sems0, sems1):
        return _conv_common(h_hbm, src_hbm, dst_hbm, zeros_hbm, xp_hbm,
                            out_hbm, sidx0, didx0, rows0, sidx1, didx1,
                            rows1, acc, sem0, sem1, sems0, sems1)

    def body_np(h_hbm, src_hbm, dst_hbm, zeros_hbm, out_hbm,
                sidx0, didx0, rows0, sidx1, didx1, rows1, acc, sem0, sem1,
                sems0, sems1):
        return _conv_common(h_hbm, src_hbm, dst_hbm, zeros_hbm, None,
                            out_hbm, sidx0, didx0, rows0, sidx1, didx1,
                            rows1, acc, sem0, sem1, sems0, sems1)

    def _conv_common(h_hbm, src_hbm, dst_hbm, zeros_hbm, xp_hbm, out_hbm,
                     sidx0, didx0, rows0, sidx1, didx1, rows1, acc,
                     sem0, sem1, sems0, sems1):
        c = lax.axis_index("c")
        sid = lax.axis_index("s")
        pltpu.sync_copy(zeros_hbm, acc.at[pl.ds(sid * rpt, rpt)])
        plsc.subcore_barrier()
        base = sid * ept
        h_view = h_hbm.at[pl.ds(c * n_in, n_in)]
        bufs = ((sidx0, didx0, rows0, sem0), (sidx1, didx1, rows1, sem1))
        sems = (sems0, sems1)

        def load_and_start(ci, b):
            si, di, ro, se = bufs[b]
            eb = base + ci * ch
            pltpu.sync_copy(src_hbm.at[pl.ds(eb, ch)], si)
            pltpu.sync_copy(dst_hbm.at[pl.ds(eb, ch)], di)
            pltpu.async_copy(h_view.at[si], ro, se)

        load_and_start(0, 0)
        load_and_start(1, 1)

        def step(i, _):
            for b in range(2):
                si, di, ro, se = bufs[b]
                pltpu.make_async_copy(h_view.at[si], ro, se).wait()
                pltpu.async_copy(ro, acc.at[di], sems[b], add=True)
            for b in range(2):
                ci = i * 2 + b
                si, di, ro, se = bufs[b]

                @pl.when(ci + 2 < steps)
                def _(b=b, ci=ci):
                    si, di, ro, se = bufs[b]
                    pltpu.make_async_copy(ro, acc.at[di], sems[b]).wait()
                    load_and_start(ci + 2, b)

            return ()

        lax.fori_loop(0, steps // 2, step, ())
        for b in range(2):
            si, di, ro, se = bufs[b]
            pltpu.make_async_copy(ro, acc.at[di], sems[b]).wait()

        if passthrough:
            w = sid * 2 + c
            nq = 2 * n_in // ch

            def pstep(i, _):
                q = i * 32 + w

                @pl.when(q < nq)
                def _():
                    r0 = q * ch
                    pltpu.sync_copy(xp_hbm.at[pl.ds(r0, ch)], rows0)
                    pltpu.sync_copy(rows0,
                                    out_hbm.at[pl.ds(2 * n_out + r0, ch)])

                return ()

            lax.fori_loop(0, (nq + 31) // 32, pstep, ())

        plsc.subcore_barrier()

        def copy_out(row0, nrows):
            pltpu.sync_copy(acc.at[pl.ds(row0, nrows)],
                            out_hbm.at[pl.ds(c * n_out + row0, nrows)])

        if lastlen == rpt:
            copy_out(sid * rpt, rpt)
        else:
            @pl.when(sid < 15)
            def _():
                copy_out(sid * rpt, rpt)

            @pl.when(sid == 15)
            def _():
                copy_out(15 * rpt, lastlen)

    return functools.partial(
        pl.kernel,
        out_type=jax.ShapeDtypeStruct((out_rows, 128), F32),
        mesh=_MESH,
        scratch_types=scratch,
    )(body_pt if passthrough else body_np)


_conv_E = _make_conv(E, N, NPAD, N, passthrough=True)
_conv_EP = _make_conv(EP, K, KPAD, K, passthrough=False)

_PCH = 40
_PNCH = N // _PCH


def _pool_body(t_hbm, s_hbm, zeros_hbm, out_hbm,
               s_v, rows0, rows1, acc0, acc1, sem):
    c = lax.axis_index("c")
    sid = lax.axis_index("s")
    rpt = KPAD // NS
    pltpu.sync_copy(zeros_hbm, acc0.at[pl.ds(sid * rpt, rpt)])
    pltpu.sync_copy(zeros_hbm, acc1.at[pl.ds(sid * rpt, rpt)])
    plsc.subcore_barrier()
    tbase = c * 2 * N

    def chunk(i, _):
        q = i * NS + sid

        @pl.when(q < _PNCH)
        def _():
            r0 = q * _PCH
            pltpu.sync_copy(s_hbm.at[pl.ds(r0, _PCH)], s_v)
            pltpu.sync_copy(t_hbm.at[pl.ds(tbase + r0, _PCH)], rows0)
            pltpu.sync_copy(t_hbm.at[pl.ds(tbase + N + r0, _PCH)], rows1)

            @pl.when(c == 0)
            def _():
                def rstep(t, _):
                    r = t // 8
                    k = t % 8
                    rows0[r, pl.ds(k * 16, 16)] = jnp.maximum(
                        rows0[r, pl.ds(k * 16, 16)], 0.0)
                    rows1[r, pl.ds(k * 16, 16)] = jnp.maximum(
                        rows1[r, pl.ds(k * 16, 16)], 0.0)
                    return ()

                lax.fori_loop(0, _PCH * 8, rstep, ())

            pltpu.sync_copy(rows0, acc0.at[s_v], add=True)
            pltpu.sync_copy(rows1, acc1.at[s_v], add=True)

        return ()

    lax.fori_loop(0, (_PNCH + NS - 1) // NS, chunk, ())
    plsc.subcore_barrier()
    lastlen = K - 15 * rpt

    def copy_out(row0, nrows):
        pltpu.sync_copy(
            acc0.at[pl.ds(row0, nrows)],
            out_hbm.at[pl.ds(row0, nrows), pl.ds(c * 256, 128)])
        pltpu.sync_copy(
            acc1.at[pl.ds(row0, nrows)],
            out_hbm.at[pl.ds(row0, nrows), pl.ds(c * 256 + 128, 128)])

    @pl.when(sid < 15)
    def _():
        copy_out(sid * rpt, rpt)

    @pl.when(sid == 15)
    def _():
        copy_out(15 * rpt, lastlen)


_pool = functools.partial(
    pl.kernel,
    out_type=jax.ShapeDtypeStruct((K, 512), F32),
    mesh=_MESH,
    scratch_types=[
        pltpu.VMEM((_PCH,), I32),
        pltpu.VMEM((_PCH, 128), F32),
        pltpu.VMEM((_PCH, 128), F32),
        pltpu.VMEM_SHARED((KPAD, 128), F32),
        pltpu.VMEM_SHARED((KPAD, 128), F32),
        pltpu.SemaphoreType.DMA,
    ],
)(_pool_body)


HW = 1572864
HA = HW + 128
NB = K * K
LASTLO = NB - HW
HPT = HW // NS
HZT = HA // NS
HCH = 80


def _hist_body(src_hbm, dst_hbm, s_hbm, zeros_hbm, out_hbm,
               sidx, didx, sval, dval, flatbuf, lidx, ones_v, flat_sp, acc,
               s_sp, sem):
    c = lax.axis_index("c")
    sid = lax.axis_index("s")
    lane = lax.iota(I32, 16)
    for g in range(HCH // 16):
        ones_v[pl.ds(g * 16, 16)] = jnp.full((16,), 1.0, F32)
    ept = E // NS
    base = sid * ept

    @pl.when(sid == 0)
    def _():
        pltpu.sync_copy(s_hbm, s_sp)

    plsc.subcore_barrier()

    def fchunk(i, _):
        eb = base + i * HCH
        pltpu.sync_copy(src_hbm.at[pl.ds(eb, HCH)], sidx)
        pltpu.sync_copy(dst_hbm.at[pl.ds(eb, HCH)], didx)
        pltpu.async_copy(s_sp.at[sidx], sval, sem).wait()
        pltpu.async_copy(s_sp.at[didx], dval, sem).wait()
        for g in range(HCH // 16):
            ss = sval[pl.ds(g * 16, 16)]
            sd = dval[pl.ds(g * 16, 16)]
            flatbuf[pl.ds(g * 16, 16)] = ss * K + sd
        pltpu.sync_copy(flatbuf, flat_sp.at[pl.ds(eb, HCH)])
        return ()

    lax.fori_loop(0, ept // HCH, fchunk, ())
    plsc.subcore_barrier()

    for p in range(8):
        pltpu.sync_copy(zeros_hbm, acc.at[pl.ds(sid * HZT, HZT)])
        plsc.subcore_barrier()
        for cc in (0, 1):
            w = 2 * p + cc
            lo = LASTLO if w == 15 else w * HW
            hi = lo + HW

            @pl.when(c == cc)
            def _(lo=lo, hi=hi):
                def schunk(i, _):
                    eb = base + i * HCH
                    pltpu.sync_copy(flat_sp.at[pl.ds(eb, HCH)], flatbuf)
                    for g in range(HCH // 16):
                        f16 = flatbuf[pl.ds(g * 16, 16)]
                        inw = (f16 >= lo) & (f16 < hi)
                        lidx[pl.ds(g * 16, 16)] = jnp.where(
                            inw, f16 - lo, HW + lane)
                    pltpu.sync_copy(ones_v, acc.at[lidx], add=True)
                    return ()

                lax.fori_loop(0, ept // HCH, schunk, ())
                pltpu.sync_copy(acc.at[pl.ds(sid * HPT, HPT)],
                                out_hbm.at[pl.ds(lo + sid * HPT, HPT)])

        plsc.subcore_barrier()


_hist = functools.partial(
    pl.kernel,
    out_type=jax.ShapeDtypeStruct((NB,), F32),
    mesh=_MESH,
    compiler_params=pltpu.CompilerParams(use_tc_tiling_on_sc=False),
    scratch_types=[
        pltpu.VMEM((HCH,), I32),
        pltpu.VMEM((HCH,), I32),
        pltpu.VMEM((HCH,), I32),
        pltpu.VMEM((HCH,), I32),
        pltpu.VMEM((HCH,), I32),
        pltpu.VMEM((HCH,), I32),
        pltpu.VMEM((HCH,), F32),
        pltpu.VMEM_SHARED((E,), I32),
        pltpu.VMEM_SHARED((HA,), F32),
        pltpu.VMEM_SHARED((N,), I32),
        pltpu.SemaphoreType.DMA,
    ],
)(_hist_body)


def kernel(x, edge_index, s, edge_index_p, noise, W_pre1, b_pre1, W_pre2,
           b_pre2, W_g1, b_g1, W_gs, b_gs, W_g2, b_g2, W_po1, b_po1,
           W_po2, b_po2):
    src, dst = edge_index[0], edge_index[1]
    src_p, dst_p = edge_index_p[0], edge_index_p[1]

    xp_t, h1_t = _t1(x, W_pre1, b_pre1.reshape(1, -1), W_pre2,
                     b_pre2.reshape(1, -1), W_g1, b_g1.reshape(1, -1))

    zeros_n = jnp.zeros((NPAD // NS, 128), F32)
    pool_t = _conv_E(h1_t, src, dst, zeros_n, xp_t)

    zeros_p = jnp.zeros((KPAD // NS, 128), F32)
    x_pool = _pool(pool_t, s, zeros_p)

    (hgs_t,) = _t3(x_pool, W_gs, b_gs.reshape(1, -1))
    zeros_k = jnp.zeros((KPAD // NS, 128), F32)
    agg_gs = _conv_EP(hgs_t, src_p, dst_p, zeros_k)

    z_mean, z, h2_t = _t4(agg_gs, x_pool, noise, W_g2, b_g2.reshape(1, -1))
    agg_g2 = _conv_EP(h2_t, src_p, dst_p, zeros_k)
    (x_lift,) = _t5(agg_g2, z, W_po1, b_po1.reshape(1, -1), W_po2,
                    b_po2.reshape(1, -1))

    zeros_h = jnp.zeros((HZT,), F32)
    a_flat = _hist(src, dst, s, zeros_h)
    a_pool = a_flat.reshape(K, K)

    return (x_lift, edge_index_p, s, x_pool, a_pool, z_mean, z_mean)


# speedup vs baseline: 1.0740x; 1.0740x over previous
import functools

import jax
import jax.numpy as jnp
from jax import lax
from jax.experimental import pallas as pl
from jax.experimental.pallas import tpu as pltpu
from jax.experimental.pallas import tpu_sc as plsc

F32 = jnp.float32
I32 = jnp.int32

N = 10000
E = 320000
K = 5000
EP = 160000
F = 128
H = 256

NS = 16
NPAD = 10240
KPAD = 5120


def _dot(a, b):
    return jnp.dot(a, b, preferred_element_type=F32)


def _half(x, j):
    return jnp.where(j == 0, x[:, :128], x[:, 128:])


def _t1_body(x_ref, w1, b1, w2, b2, wg, bg, xp_ref, h1_ref):
    j = pl.program_id(0)
    h = jnp.maximum(_dot(x_ref[...], w1[...]) + b1[...], 0.0)
    xp = jnp.maximum(_dot(h, w2[...]) + b2[...], 0.0)
    xp_ref[...] = _half(xp, j)
    h1_ref[...] = _dot(xp, wg[...]) + bg[...]


def _t1(x, W1, b1, W2, b2, Wg, bg):
    BM = 1000
    g = N // BM
    return pl.pallas_call(
        _t1_body,
        grid=(2, g),
        in_specs=[
            pl.BlockSpec((BM, F), lambda j, i: (i, 0)),
            pl.BlockSpec((F, H), lambda j, i: (0, 0)),
            pl.BlockSpec((1, H), lambda j, i: (0, 0)),
            pl.BlockSpec((H, H), lambda j, i: (0, 0)),
            pl.BlockSpec((1, H), lambda j, i: (0, 0)),
            pl.BlockSpec((H, 128), lambda j, i: (0, j)),
            pl.BlockSpec((1, 128), lambda j, i: (0, j)),
        ],
        out_specs=[
            pl.BlockSpec((BM, 128), lambda j, i: (j * g + i, 0)),
            pl.BlockSpec((BM, 128), lambda j, i: (j * g + i, 0)),
        ],
        out_shape=[
            jax.ShapeDtypeStruct((2 * N, 128), F32),
            jax.ShapeDtypeStruct((2 * N, 128), F32),
        ],
    )(x, W1, b1, W2, b2, Wg, bg)


def _t3_body(xp_ref, wgs, bgs, h_ref):
    h_ref[...] = _dot(xp_ref[...], wgs[...]) + bgs[...]


def _t3(x_pool, Wgs, bgs):
    BM = 1000
    g = K // BM
    return pl.pallas_call(
        _t3_body,
        grid=(2, g),
        in_specs=[
            pl.BlockSpec((BM, 2 * H), lambda j, i: (i, 0)),
            pl.BlockSpec((2 * H, 128), lambda j, i: (0, j)),
            pl.BlockSpec((1, 128), lambda j, i: (0, j)),
        ],
        out_specs=[pl.BlockSpec((BM, 128), lambda j, i: (j * g + i, 0))],
        out_shape=[jax.ShapeDtypeStruct((2 * K, 128), F32)],
    )(x_pool, Wgs, bgs)


def _t4_body(aga_ref, agb_ref, xp_ref, noise_ref, wg2, bg2, zm_ref, z_ref,
             h2_ref):
    gs = jnp.concatenate(
        [jnp.maximum(aga_ref[...], 0.0), jnp.maximum(agb_ref[...], 0.0)],
        axis=1)
    zm = jnp.concatenate([gs, xp_ref[...]], axis=1)
    z = zm + noise_ref[...] * jnp.exp(zm)
    zm_ref[...] = zm
    z_ref[...] = z
    h2_ref[...] = _dot(z, wg2[...]) + bg2[...]


def _t4(agg, x_pool, noise, Wg2, bg2):
    BM = 1000
    g = K // BM
    return pl.pallas_call(
        _t4_body,
        grid=(2, g),
        in_specs=[
            pl.BlockSpec((BM, 128), lambda j, i: (i, 0)),
            pl.BlockSpec((BM, 128), lambda j, i: (g + i, 0)),
            pl.BlockSpec((BM, 2 * H), lambda j, i: (i, 0)),
            pl.BlockSpec((BM, 3 * H), lambda j, i: (i, 0)),
            pl.BlockSpec((3 * H, 128), lambda j, i: (0, j)),
            pl.BlockSpec((1, 128), lambda j, i: (0, j)),
        ],
        out_specs=[
            pl.BlockSpec((BM, 3 * H), lambda j, i: (i, 0)),
            pl.BlockSpec((BM, 3 * H), lambda j, i: (i, 0)),
            pl.BlockSpec((BM, 128), lambda j, i: (j * g + i, 0)),
        ],
        out_shape=[
            jax.ShapeDtypeStruct((K, 3 * H), F32),
            jax.ShapeDtypeStruct((K, 3 * H), F32),
            jax.ShapeDtypeStruct((2 * K, 128), F32),
        ],
    )(agg, agg, x_pool, noise, Wg2, bg2)


def _t5_body(aga_ref, agb_ref, z_ref, wp1, bp1, wp2, bp2, xl_ref):
    g2 = jnp.concatenate(
        [jnp.maximum(aga_ref[...], 0.0), jnp.maximum(agb_ref[...], 0.0)],
        axis=1)
    zsk = jnp.concatenate([g2, z_ref[...]], axis=1)
    hh = jnp.maximum(_dot(zsk, wp1[...]) + bp1[...], 0.0)
    xl_ref[...] = _dot(hh, wp2[...]) + bp2[...]


def _t5(agg, z, Wp1, bp1, Wp2, bp2):
    BM = 1000
    g = K // BM
    return pl.pallas_call(
        _t5_body,
        grid=(g,),
        in_specs=[
            pl.BlockSpec((BM, 128), lambda i: (i, 0)),
            pl.BlockSpec((BM, 128), lambda i: (K // 1000 + i, 0)),
            pl.BlockSpec((BM, 3 * H), lambda i: (i, 0)),
            pl.BlockSpec((4 * H, H), lambda i: (0, 0)),
            pl.BlockSpec((1, H), lambda i: (0, 0)),
            pl.BlockSpec((H, F), lambda i: (0, 0)),
            pl.BlockSpec((1, F), lambda i: (0, 0)),
        ],
        out_specs=[pl.BlockSpec((BM, F), lambda i: (i, 0))],
        out_shape=[jax.ShapeDtypeStruct((K, F), F32)],
    )(agg, agg, z, Wp1, bp1, Wp2, bp2)


_MESH = plsc.VectorSubcoreMesh(core_axis_name="c", subcore_axis_name="s")


def _make_conv(num_edges, n_in, n_pad, n_out, passthrough, ch=40):
    ept = num_edges // NS
    rpt = n_pad // NS
    steps = ept // ch
    lastlen = n_out - 15 * rpt
    out_rows = (4 if passthrough else 2) * n_out

    scratch = [
        pltpu.VMEM((ch,), I32),
        pltpu.VMEM((ch,), I32),
        pltpu.VMEM((ch, 128), F32),
        pltpu.VMEM((ch,), I32),
        pltpu.VMEM((ch,), I32),
        pltpu.VMEM((ch, 128), F32),
        pltpu.VMEM_SHARED((n_pad, 128), F32),
        pltpu.SemaphoreType.DMA,
        pltpu.SemaphoreType.DMA,
    ]

    def body_pt(h_hbm, src_hbm, dst_hbm, zeros_hbm, xp_hbm, out_hbm,
                sidx0, didx0, rows0, sidx1, didx1, rows1, acc, sem0, sem1):
        return _conv_common(h_hbm, src_hbm, dst_hbm, zeros_hbm, xp_hbm,
                            out_hbm, sidx0, didx0, rows0, sidx1, didx1,
                            rows1, acc, sem0, sem1)

    def body_np(h_hbm, src_hbm, dst_hbm, zeros_hbm, out_hbm,
                sidx0, didx0, rows0, sidx1, didx1, rows1, acc, sem0, sem1):
        return _conv_common(h_hbm, src_hbm, dst_hbm, zeros_hbm, None,
                            out_hbm, sidx0, didx0, rows0, sidx1, didx1,
                            rows1, acc, sem0, sem1)

    def _conv_common(h_hbm, src_hbm, dst_hbm, zeros_hbm, xp_hbm, out_hbm,
                     sidx0, didx0, rows0, sidx1, didx1, rows1, acc,
                     sem0, sem1):
        c = lax.axis_index("c")
        sid = lax.axis_index("s")
        pltpu.sync_copy(zeros_hbm, acc.at[pl.ds(sid * rpt, rpt)])
        plsc.subcore_barrier()
        base = sid * ept
        h_view = h_hbm.at[pl.ds(c * n_in, n_in)]
        bufs = ((sidx0, didx0, rows0, sem0), (sidx1, didx1, rows1, sem1))

        def load_and_start(ci, b):
            si, di, ro, se = bufs[b]
            eb = base + ci * ch
            pltpu.sync_copy(src_hbm.at[pl.ds(eb, ch)], si)
            pltpu.sync_copy(dst_hbm.at[pl.ds(eb, ch)], di)
            pltpu.async_copy(h_view.at[si], ro, se)

        load_and_start(0, 0)
        load_and_start(1, 1)

        def step(i, _):
            for b in range(2):
                ci = i * 2 + b
                si, di, ro, se = bufs[b]
                pltpu.make_async_copy(h_view.at[si], ro, se).wait()
                pltpu.sync_copy(ro, acc.at[di], add=True)

                @pl.when(ci + 2 < steps)
                def _(b=b, ci=ci):
                    load_and_start(ci + 2, b)

            return ()

        lax.fori_loop(0, steps // 2, step, ())

        if passthrough:
            w = sid * 2 + c
            nq = 2 * n_in // ch

            def pstep(i, _):
                q = i * 32 + w

                @pl.when(q < nq)
                def _():
                    r0 = q * ch
                    pltpu.sync_copy(xp_hbm.at[pl.ds(r0, ch)], rows0)
                    pltpu.sync_copy(rows0,
                                    out_hbm.at[pl.ds(2 * n_out + r0, ch)])

                return ()

            lax.fori_loop(0, (nq + 31) // 32, pstep, ())

        plsc.subcore_barrier()

        def copy_out(row0, nrows):
            pltpu.sync_copy(acc.at[pl.ds(row0, nrows)],
                            out_hbm.at[pl.ds(c * n_out + row0, nrows)])

        if lastlen == rpt:
            copy_out(sid * rpt, rpt)
        else:
            @pl.when(sid < 15)
            def _():
                copy_out(sid * rpt, rpt)

            @pl.when(sid == 15)
            def _():
                copy_out(15 * rpt, lastlen)

    return functools.partial(
        pl.kernel,
        out_type=jax.ShapeDtypeStruct((out_rows, 128), F32),
        mesh=_MESH,
        scratch_types=scratch,
    )(body_pt if passthrough else body_np)


_conv_E = _make_conv(E, N, NPAD, N, passthrough=True)
_conv_EP = _make_conv(EP, K, KPAD, K, passthrough=False)

_PCH = 40
_PNCH = N // _PCH


def _pool_body(t_hbm, s_hbm, zeros_hbm, out_hbm,
               s_v, rows0, rows1, acc0, acc1, sem):
    c = lax.axis_index("c")
    sid = lax.axis_index("s")
    rpt = KPAD // NS
    pltpu.sync_copy(zeros_hbm, acc0.at[pl.ds(sid * rpt, rpt)])
    pltpu.sync_copy(zeros_hbm, acc1.at[pl.ds(sid * rpt, rpt)])
    plsc.subcore_barrier()
    tbase = c * 2 * N

    def chunk(i, _):
        q = i * NS + sid

        @pl.when(q < _PNCH)
        def _():
            r0 = q * _PCH
            pltpu.sync_copy(s_hbm.at[pl.ds(r0, _PCH)], s_v)
            pltpu.sync_copy(t_hbm.at[pl.ds(tbase + r0, _PCH)], rows0)
            pltpu.sync_copy(t_hbm.at[pl.ds(tbase + N + r0, _PCH)], rows1)

            @pl.when(c == 0)
            def _():
                def rstep(t, _):
                    r = t // 8
                    k = t % 8
                    rows0[r, pl.ds(k * 16, 16)] = jnp.maximum(
                        rows0[r, pl.ds(k * 16, 16)], 0.0)
                    rows1[r, pl.ds(k * 16, 16)] = jnp.maximum(
                        rows1[r, pl.ds(k * 16, 16)], 0.0)
                    return ()

                lax.fori_loop(0, _PCH * 8, rstep, ())

            pltpu.sync_copy(rows0, acc0.at[s_v], add=True)
            pltpu.sync_copy(rows1, acc1.at[s_v], add=True)

        return ()

    lax.fori_loop(0, (_PNCH + NS - 1) // NS, chunk, ())
    plsc.subcore_barrier()
    lastlen = K - 15 * rpt

    def copy_out(row0, nrows):
        pltpu.sync_copy(
            acc0.at[pl.ds(row0, nrows)],
            out_hbm.at[pl.ds(row0, nrows), pl.ds(c * 256, 128)])
        pltpu.sync_copy(
            acc1.at[pl.ds(row0, nrows)],
            out_hbm.at[pl.ds(row0, nrows), pl.ds(c * 256 + 128, 128)])

    @pl.when(sid < 15)
    def _():
        copy_out(sid * rpt, rpt)

    @pl.when(sid == 15)
    def _():
        copy_out(15 * rpt, lastlen)


_pool = functools.partial(
    pl.kernel,
    out_type=jax.ShapeDtypeStruct((K, 512), F32),
    mesh=_MESH,
    scratch_types=[
        pltpu.VMEM((_PCH,), I32),
        pltpu.VMEM((_PCH, 128), F32),
        pltpu.VMEM((_PCH, 128), F32),
        pltpu.VMEM_SHARED((KPAD, 128), F32),
        pltpu.VMEM_SHARED((KPAD, 128), F32),
        pltpu.SemaphoreType.DMA,
    ],
)(_pool_body)


HW = 1572864
HA = HW + 128
NB = K * K
LASTLO = NB - HW
HPT = HW // NS
HZT = HA // NS
HCH = 80


def _hist_body(src_hbm, dst_hbm, s_hbm, zeros_hbm, out_hbm,
               sidx, didx, sval, dval, flatbuf, lidx, ones_v, flat_sp, acc,
               s_sp, sem):
    c = lax.axis_index("c")
    sid = lax.axis_index("s")
    lane = lax.iota(I32, 16)
    for g in range(HCH // 16):
        ones_v[pl.ds(g * 16, 16)] = jnp.full((16,), 1.0, F32)
    ept = E // NS
    base = sid * ept

    @pl.when(sid == 0)
    def _():
        pltpu.sync_copy(s_hbm, s_sp)

    plsc.subcore_barrier()

    def fchunk(i, _):
        eb = base + i * HCH
        pltpu.sync_copy(src_hbm.at[pl.ds(eb, HCH)], sidx)
        pltpu.sync_copy(dst_hbm.at[pl.ds(eb, HCH)], didx)
        pltpu.async_copy(s_sp.at[sidx], sval, sem).wait()
        pltpu.async_copy(s_sp.at[didx], dval, sem).wait()
        for g in range(HCH // 16):
            ss = sval[pl.ds(g * 16, 16)]
            sd = dval[pl.ds(g * 16, 16)]
            flatbuf[pl.ds(g * 16, 16)] = ss * K + sd
        pltpu.sync_copy(flatbuf, flat_sp.at[pl.ds(eb, HCH)])
        return ()

    lax.fori_loop(0, ept // HCH, fchunk, ())
    plsc.subcore_barrier()

    for p in range(8):
        pltpu.sync_copy(zeros_hbm, acc.at[pl.ds(sid * HZT, HZT)])
        plsc.subcore_barrier()
        for cc in (0, 1):
            w = 2 * p + cc
            lo = LASTLO if w == 15 else w * HW
            hi = lo + HW

            @pl.when(c == cc)
            def _(lo=lo, hi=hi):
                def schunk(i, _):
                    eb = base + i * HCH
                    pltpu.sync_copy(flat_sp.at[pl.ds(eb, HCH)], flatbuf)
                    for g in range(HCH // 16):
                        f16 = flatbuf[pl.ds(g * 16, 16)]
                        inw = (f16 >= lo) & (f16 < hi)
                        lidx[pl.ds(g * 16, 16)] = jnp.where(
                            inw, f16 - lo, HW + lane)
                    pltpu.sync_copy(ones_v, acc.at[lidx], add=True)
                    return ()

                lax.fori_loop(0, ept // HCH, schunk, ())
                pltpu.sync_copy(acc.at[pl.ds(sid * HPT, HPT)],
                                out_hbm.at[pl.ds(lo + sid * HPT, HPT)])

        plsc.subcore_barrier()


_hist = functools.partial(
    pl.kernel,
    out_type=jax.ShapeDtypeStruct((NB,), F32),
    mesh=_MESH,
    compiler_params=pltpu.CompilerParams(use_tc_tiling_on_sc=False),
    scratch_types=[
        pltpu.VMEM((HCH,), I32),
        pltpu.VMEM((HCH,), I32),
        pltpu.VMEM((HCH,), I32),
        pltpu.VMEM((HCH,), I32),
        pltpu.VMEM((HCH,), I32),
        pltpu.VMEM((HCH,), I32),
        pltpu.VMEM((HCH,), F32),
        pltpu.VMEM_SHARED((E,), I32),
        pltpu.VMEM_SHARED((HA,), F32),
        pltpu.VMEM_SHARED((N,), I32),
        pltpu.SemaphoreType.DMA,
    ],
)(_hist_body)


def kernel(x, edge_index, s, edge_index_p, noise, W_pre1, b_pre1, W_pre2,
           b_pre2, W_g1, b_g1, W_gs, b_gs, W_g2, b_g2, W_po1, b_po1,
           W_po2, b_po2):
    src, dst = edge_index[0], edge_index[1]
    src_p, dst_p = edge_index_p[0], edge_index_p[1]

    xp_t, h1_t = _t1(x, W_pre1, b_pre1.reshape(1, -1), W_pre2,
                     b_pre2.reshape(1, -1), W_g1, b_g1.reshape(1, -1))

    zeros_n = jnp.zeros((NPAD // NS, 128), F32)
    pool_t = _conv_E(h1_t, src, dst, zeros_n, xp_t)

    zeros_p = jnp.zeros((KPAD // NS, 128), F32)
    x_pool = _pool(pool_t, s, zeros_p)

    (hgs_t,) = _t3(x_pool, W_gs, b_gs.reshape(1, -1))
    zeros_k = jnp.zeros((KPAD // NS, 128), F32)
    agg_gs = _conv_EP(hgs_t, src_p, dst_p, zeros_k)

    z_mean, z, h2_t = _t4(agg_gs, x_pool, noise, W_g2, b_g2.reshape(1, -1))
    agg_g2 = _conv_EP(h2_t, src_p, dst_p, zeros_k)
    (x_lift,) = _t5(agg_g2, z, W_po1, b_po1.reshape(1, -1), W_po2,
                    b_po2.reshape(1, -1))

    zeros_h = jnp.zeros((HZT,), F32)
    a_flat = _hist(src, dst, s, zeros_h)
    a_pool = a_flat.reshape(K, K)

    return (x_lift, edge_index_p, s, x_pool, a_pool, z_mean, z_mean)
